# Initial kernel scaffold; baseline (speedup 1.0000x reference)
import jax
import jax.numpy as jnp
from jax.experimental import pallas as pl


def kernel(x, codebook):
    raise NotImplementedError("write your pallas kernel here")


# speedup vs baseline: 1.2786x; 1.2786x over previous
import functools

import jax
import jax.numpy as jnp
from jax import lax
from jax.experimental import pallas as pl
from jax.experimental.pallas import tpu as pltpu
from jax.experimental.pallas import tpu_sc as plsc

N_CODES = 8192
CODE_DIM = 256
N_ROWS = 16384

BN = 512
K_CHUNKS = ((0, 2736), (2736, 2736), (5472, 2720))


def _argmin_body(xm2_ref, cb_ref, xsq_ref, csq_ref, idx_ref):
    acc_v = None
    acc_i = None
    for lo, sz in K_CHUNKS:
        dots = lax.dot_general(
            xm2_ref[...], cb_ref[pl.ds(lo, sz), :], (((1,), (1,)), ((), ())),
            preferred_element_type=jnp.float32,
        )
        dist = (xsq_ref[...] + dots) + csq_ref[:, pl.ds(lo, sz)]
        v = jnp.min(dist, axis=1, keepdims=True)
        cols = lax.broadcasted_iota(jnp.int32, (BN, sz), 1)
        i = jnp.min(
            jnp.where(dist == v, cols, jnp.int32(N_CODES)), axis=1, keepdims=True
        ) + lo
        if acc_v is None:
            acc_v = v.astype(jnp.bfloat16).astype(jnp.float32)
            acc_i = i
        else:
            keep = (acc_v < v) | ((acc_v == v) & (acc_i < i))
            acc_i = jnp.where(keep, acc_i, i)
            acc_v = jnp.where(keep, acc_v, v).astype(jnp.bfloat16).astype(jnp.float32)
    idx_ref[...] = acc_i.reshape(1, 1, BN)


def _quantize_tc(xm2_bf, cb_bf, xsq, csq):
    n_tiles = N_ROWS // BN
    idx3 = pl.pallas_call(
        _argmin_body,
        grid=(n_tiles,),
        in_specs=[
            pl.BlockSpec((BN, CODE_DIM), lambda n: (n, 0)),
            pl.BlockSpec((N_CODES, CODE_DIM), lambda n: (0, 0)),
            pl.BlockSpec((BN, 1), lambda n: (n, 0)),
            pl.BlockSpec((1, N_CODES), lambda n: (0, 0)),
        ],
        out_specs=pl.BlockSpec((1, 1, BN), lambda n: (n, 0, 0)),
        out_shape=jax.ShapeDtypeStruct((n_tiles, 1, BN), jnp.int32),
        compiler_params=pltpu.CompilerParams(
            dimension_semantics=("parallel",),
        ),
    )(xm2_bf, cb_bf, xsq, csq)
    return idx3.reshape(N_ROWS)


_SC_NUM_CORES = 2
_SC_NUM_SUBCORES = 16
_NW = _SC_NUM_CORES * _SC_NUM_SUBCORES
_ROWS_PER_W = N_ROWS // _NW
_CHUNK = 128
_N_CHUNKS = _ROWS_PER_W // _CHUNK


def _gather_body(cb_hbm, idx_hbm, out_hbm, idx_v, rows_v, sem):
    wid = lax.axis_index("s") * _SC_NUM_CORES + lax.axis_index("c")
    base = wid * _ROWS_PER_W
    for c in range(_N_CHUNKS):
        off = base + c * _CHUNK
        pltpu.sync_copy(idx_hbm.at[pl.ds(off, _CHUNK)], idx_v)
        pltpu.async_copy(cb_hbm.at[idx_v], rows_v, sem).wait()
        pltpu.sync_copy(rows_v, out_hbm.at[pl.ds(off, _CHUNK)])


@functools.cache
def _gather_sc():
    return pl.kernel(
        _gather_body,
        out_type=jax.ShapeDtypeStruct((N_ROWS, CODE_DIM), jnp.float32),
        mesh=plsc.VectorSubcoreMesh(
            core_axis_name="c",
            subcore_axis_name="s",
            num_cores=_SC_NUM_CORES,
            num_subcores=_SC_NUM_SUBCORES,
        ),
        scratch_types=[
            pltpu.VMEM((_CHUNK,), jnp.int32),
            pltpu.VMEM((_CHUNK, CODE_DIM), jnp.float32),
            pltpu.SemaphoreType.DMA,
        ],
    )


def kernel(x, codebook):
    flat = x.reshape(-1, CODE_DIM)
    xsq = jnp.sum(flat * flat, axis=1, keepdims=True)
    csq = jnp.sum(codebook * codebook, axis=1)[None, :]
    xm2_bf = (-2.0 * flat).astype(jnp.bfloat16)
    cb_bf = codebook.astype(jnp.bfloat16)
    idx = _quantize_tc(xm2_bf, cb_bf, xsq, csq)
    codes = _gather_sc()(codebook, idx)
    return (x, codes.reshape(x.shape))


# speedup vs baseline: 1.4251x; 1.1146x over previous
import functools

import jax
import jax.numpy as jnp
from jax import lax
from jax.experimental import pallas as pl
from jax.experimental.pallas import tpu as pltpu
from jax.experimental.pallas import tpu_sc as plsc

N_CODES = 8192
CODE_DIM = 256
N_ROWS = 16384

BN = 1024
K_CHUNK_REAL = (2736, 2736, 2720)
K_CHUNK_PAD = 2816
K_PAD_TOTAL = K_CHUNK_PAD * 3


def _argmin_body(xm2_ref, cb_ref, xsq_ref, csq_ref, cols_ref, idx_ref):
    acc_v = None
    acc_i = None
    base = 0
    for c, real_sz in enumerate(K_CHUNK_REAL):
        lo = c * K_CHUNK_PAD
        dots = lax.dot_general(
            xm2_ref[...], cb_ref[pl.ds(lo, K_CHUNK_PAD), :],
            (((1,), (1,)), ((), ())),
            preferred_element_type=jnp.float32,
        )
        dist = (xsq_ref[...] + dots) + csq_ref[:, pl.ds(lo, K_CHUNK_PAD)]
        v = jnp.min(dist, axis=1, keepdims=True)
        i_f = jnp.min(
            jnp.where(dist == v, cols_ref[...], jnp.float32(K_PAD_TOTAL)),
            axis=1, keepdims=True,
        )
        i = i_f.astype(jnp.int32) + base
        if acc_v is None:
            acc_v = v.astype(jnp.bfloat16).astype(jnp.float32)
            acc_i = i
        else:
            keep = (acc_v < v) | ((acc_v == v) & (acc_i < i))
            acc_i = jnp.where(keep, acc_i, i)
            acc_v = jnp.where(keep, acc_v, v).astype(jnp.bfloat16).astype(jnp.float32)
        base += real_sz
    idx_ref[...] = acc_i.reshape(1, 1, BN)


def _quantize_tc(xm2_bf, cb_bf, xsq, csq, cols_row):
    n_tiles = N_ROWS // BN
    idx3 = pl.pallas_call(
        _argmin_body,
        grid=(n_tiles,),
        in_specs=[
            pl.BlockSpec((BN, CODE_DIM), lambda n: (n, 0)),
            pl.BlockSpec((K_PAD_TOTAL, CODE_DIM), lambda n: (0, 0)),
            pl.BlockSpec((BN, 1), lambda n: (n, 0)),
            pl.BlockSpec((1, K_PAD_TOTAL), lambda n: (0, 0)),
            pl.BlockSpec((1, K_CHUNK_PAD), lambda n: (0, 0)),
        ],
        out_specs=pl.BlockSpec((1, 1, BN), lambda n: (n, 0, 0)),
        out_shape=jax.ShapeDtypeStruct((n_tiles, 1, BN), jnp.int32),
        compiler_params=pltpu.CompilerParams(
            dimension_semantics=("parallel",),
        ),
    )(xm2_bf, cb_bf, xsq, csq, cols_row)
    return idx3.reshape(N_ROWS)


_SC_NUM_CORES = 2
_SC_NUM_SUBCORES = 16
_NW = _SC_NUM_CORES * _SC_NUM_SUBCORES
_ROWS_PER_W = N_ROWS // _NW
_CHUNK = 128
_N_CHUNKS = _ROWS_PER_W // _CHUNK


def _gather_body(cb_hbm, idx_hbm, out_hbm, idx_v, rows_v, sem):
    wid = lax.axis_index("s") * _SC_NUM_CORES + lax.axis_index("c")
    base = wid * _ROWS_PER_W
    for c in range(_N_CHUNKS):
        off = base + c * _CHUNK
        pltpu.sync_copy(idx_hbm.at[pl.ds(off, _CHUNK)], idx_v)
        pltpu.async_copy(cb_hbm.at[idx_v], rows_v, sem).wait()
        pltpu.sync_copy(rows_v, out_hbm.at[pl.ds(off, _CHUNK)])


@functools.cache
def _gather_sc():
    return pl.kernel(
        _gather_body,
        out_type=jax.ShapeDtypeStruct((N_ROWS, CODE_DIM), jnp.float32),
        mesh=plsc.VectorSubcoreMesh(
            core_axis_name="c",
            subcore_axis_name="s",
            num_cores=_SC_NUM_CORES,
            num_subcores=_SC_NUM_SUBCORES,
        ),
        scratch_types=[
            pltpu.VMEM((_CHUNK,), jnp.int32),
            pltpu.VMEM((_CHUNK, CODE_DIM), jnp.float32),
            pltpu.SemaphoreType.DMA,
        ],
    )


def kernel(x, codebook):
    flat = x.reshape(-1, CODE_DIM)
    xsq = jnp.sum(flat * flat, axis=1, keepdims=True)
    csq = jnp.sum(codebook * codebook, axis=1)
    xm2_bf = (-2.0 * flat).astype(jnp.bfloat16)
    cb_bf = codebook.astype(jnp.bfloat16)
    cb_pad = jnp.zeros((K_PAD_TOTAL, CODE_DIM), jnp.bfloat16)
    csq_pad = jnp.full((K_PAD_TOTAL,), jnp.inf, jnp.float32)
    base = 0
    for c, real_sz in enumerate(K_CHUNK_REAL):
        lo = c * K_CHUNK_PAD
        cb_pad = lax.dynamic_update_slice(
            cb_pad, lax.dynamic_slice(cb_bf, (base, 0), (real_sz, CODE_DIM)), (lo, 0))
        csq_pad = lax.dynamic_update_slice(
            csq_pad, lax.dynamic_slice(csq, (base,), (real_sz,)), (lo,))
        base += real_sz
    cols_row = lax.iota(jnp.float32, K_CHUNK_PAD)[None, :]
    idx = _quantize_tc(xm2_bf, cb_pad, xsq, csq_pad[None, :], cols_row)
    codes = _gather_sc()(codebook, idx)
    return (x, codes.reshape(x.shape))


# speedup vs baseline: 1.5066x; 1.0572x over previous
import functools

import jax
import jax.numpy as jnp
from jax import lax
from jax.experimental import pallas as pl
from jax.experimental.pallas import tpu as pltpu
from jax.experimental.pallas import tpu_sc as plsc

N_CODES = 8192
CODE_DIM = 256
N_ROWS = 16384

BN = 1024
K_CHUNK_REAL = (2736, 2736, 2720)
K_CHUNK_PAD = 2816
K_PAD_TOTAL = K_CHUNK_PAD * 3


LANES = 128


def _argmin_body(x_ref, cb_ref, xsq_ref, csq_ref, cols_ref, idx_ref):
    xm2 = (-2.0 * x_ref[...]).astype(jnp.bfloat16)
    xsq = xsq_ref[...]
    acc_v = None
    acc_i = None
    base = 0
    for c, real_sz in enumerate(K_CHUNK_REAL):
        lo = c * K_CHUNK_PAD
        dots = lax.dot_general(
            xm2, cb_ref[pl.ds(lo, K_CHUNK_PAD), :],
            (((1,), (1,)), ((), ())),
            preferred_element_type=jnp.float32,
        )
        run_v = jnp.full((BN, LANES), jnp.inf, jnp.float32)
        run_c = jnp.zeros((BN, LANES), jnp.float32)
        for k in range(K_CHUNK_PAD // LANES):
            s0, s1 = k * LANES, (k + 1) * LANES
            d = (xsq + dots[:, s0:s1]) + csq_ref[:, pl.ds(lo + s0, LANES)]
            m = d < run_v
            run_v = jnp.where(m, d, run_v)
            run_c = jnp.where(m, cols_ref[:, pl.ds(s0, LANES)], run_c)
        v = jnp.min(run_v, axis=1, keepdims=True)
        i_f = jnp.min(
            jnp.where(run_v == v, run_c, jnp.float32(K_PAD_TOTAL)),
            axis=1, keepdims=True,
        )
        i = i_f.astype(jnp.int32) + base
        if acc_v is None:
            acc_v = v.astype(jnp.bfloat16).astype(jnp.float32)
            acc_i = i
        else:
            keep = (acc_v < v) | ((acc_v == v) & (acc_i < i))
            acc_i = jnp.where(keep, acc_i, i)
            acc_v = jnp.where(keep, acc_v, v).astype(jnp.bfloat16).astype(jnp.float32)
        base += real_sz
    idx_ref[...] = acc_i.reshape(1, 1, BN)


def _quantize_tc(flat, cb_bf, xsq, csq, cols_row):
    n_tiles = N_ROWS // BN
    idx3 = pl.pallas_call(
        _argmin_body,
        grid=(n_tiles,),
        in_specs=[
            pl.BlockSpec((BN, CODE_DIM), lambda n: (n, 0)),
            pl.BlockSpec((K_PAD_TOTAL, CODE_DIM), lambda n: (0, 0)),
            pl.BlockSpec((BN, 1), lambda n: (n, 0)),
            pl.BlockSpec((1, K_PAD_TOTAL), lambda n: (0, 0)),
            pl.BlockSpec((1, K_CHUNK_PAD), lambda n: (0, 0)),
        ],
        out_specs=pl.BlockSpec((1, 1, BN), lambda n: (n, 0, 0)),
        out_shape=jax.ShapeDtypeStruct((n_tiles, 1, BN), jnp.int32),
        compiler_params=pltpu.CompilerParams(
            dimension_semantics=("parallel",),
        ),
    )(flat, cb_bf, xsq, csq, cols_row)
    return idx3.reshape(N_ROWS)


_SC_NUM_CORES = 2
_SC_NUM_SUBCORES = 16
_NW = _SC_NUM_CORES * _SC_NUM_SUBCORES
_ROWS_PER_W = N_ROWS // _NW
_CHUNK = 128
_N_CHUNKS = _ROWS_PER_W // _CHUNK


def _gather_body(cb_hbm, idx_hbm, out_hbm, idx_v, rows_v, sem):
    wid = lax.axis_index("s") * _SC_NUM_CORES + lax.axis_index("c")
    base = wid * _ROWS_PER_W
    for c in range(_N_CHUNKS):
        off = base + c * _CHUNK
        pltpu.sync_copy(idx_hbm.at[pl.ds(off, _CHUNK)], idx_v)
        pltpu.async_copy(cb_hbm.at[idx_v], rows_v, sem).wait()
        pltpu.sync_copy(rows_v, out_hbm.at[pl.ds(off, _CHUNK)])


@functools.cache
def _gather_sc():
    return pl.kernel(
        _gather_body,
        out_type=jax.ShapeDtypeStruct((N_ROWS, CODE_DIM), jnp.float32),
        mesh=plsc.VectorSubcoreMesh(
            core_axis_name="c",
            subcore_axis_name="s",
            num_cores=_SC_NUM_CORES,
            num_subcores=_SC_NUM_SUBCORES,
        ),
        scratch_types=[
            pltpu.VMEM((_CHUNK,), jnp.int32),
            pltpu.VMEM((_CHUNK, CODE_DIM), jnp.float32),
            pltpu.SemaphoreType.DMA,
        ],
    )


def kernel(x, codebook):
    flat = x.reshape(-1, CODE_DIM)
    xsq = jnp.sum(flat * flat, axis=1, keepdims=True)
    csq = jnp.sum(codebook * codebook, axis=1)
    cb_bf = codebook.astype(jnp.bfloat16)
    cb_pad = jnp.zeros((K_PAD_TOTAL, CODE_DIM), jnp.bfloat16)
    csq_pad = jnp.full((K_PAD_TOTAL,), jnp.inf, jnp.float32)
    base = 0
    for c, real_sz in enumerate(K_CHUNK_REAL):
        lo = c * K_CHUNK_PAD
        cb_pad = lax.dynamic_update_slice(
            cb_pad, lax.dynamic_slice(cb_bf, (base, 0), (real_sz, CODE_DIM)), (lo, 0))
        csq_pad = lax.dynamic_update_slice(
            csq_pad, lax.dynamic_slice(csq, (base,), (real_sz,)), (lo,))
        base += real_sz
    cols_row = lax.iota(jnp.float32, K_CHUNK_PAD)[None, :]
    idx = _quantize_tc(flat, cb_pad, xsq, csq_pad[None, :], cols_row)
    codes = _gather_sc()(codebook, idx)
    return (x, codes.reshape(x.shape))


# speedup vs baseline: 1.5361x; 1.0196x over previous
import functools

import jax
import jax.numpy as jnp
from jax import lax
from jax.experimental import pallas as pl
from jax.experimental.pallas import tpu as pltpu
from jax.experimental.pallas import tpu_sc as plsc

N_CODES = 8192
CODE_DIM = 256
N_ROWS = 16384

BN = 1024
K_CHUNK_REAL = (2736, 2736, 2720)
K_CHUNK_PAD = 2816
K_PAD_TOTAL = K_CHUNK_PAD * 3


LANES = 128
ROWB = 128


def _argmin_body(x_ref, cb_ref, xsq_ref, csq_ref, cols_ref, idx_ref):
    xm2 = (-2.0 * x_ref[...]).astype(jnp.bfloat16)
    xsq = xsq_ref[...]
    acc_v = None
    acc_i = None
    base = 0
    for c, real_sz in enumerate(K_CHUNK_REAL):
        lo = c * K_CHUNK_PAD
        dots = lax.dot_general(
            xm2, cb_ref[pl.ds(lo, K_CHUNK_PAD), :],
            (((1,), (1,)), ((), ())),
            preferred_element_type=jnp.float32,
        )
        v_blocks = []
        i_blocks = []
        for r0 in range(0, BN, ROWB):
            run_v = jnp.full((ROWB, LANES), jnp.inf, jnp.float32)
            run_c = jnp.zeros((ROWB, LANES), jnp.float32)
            xsq_b = xsq[r0:r0 + ROWB, :]
            for k in range(K_CHUNK_PAD // LANES):
                s0, s1 = k * LANES, (k + 1) * LANES
                d = (xsq_b + dots[r0:r0 + ROWB, s0:s1]) \
                    + csq_ref[:, pl.ds(lo + s0, LANES)]
                m = d < run_v
                run_v = jnp.where(m, d, run_v)
                run_c = jnp.where(m, cols_ref[:, pl.ds(s0, LANES)], run_c)
            vb = jnp.min(run_v, axis=1, keepdims=True)
            ib = jnp.min(
                jnp.where(run_v == vb, run_c, jnp.float32(K_PAD_TOTAL)),
                axis=1, keepdims=True,
            )
            v_blocks.append(vb)
            i_blocks.append(ib)
        v = jnp.concatenate(v_blocks, axis=0)
        i_f = jnp.concatenate(i_blocks, axis=0)
        i = i_f.astype(jnp.int32) + base
        if acc_v is None:
            acc_v = v.astype(jnp.bfloat16).astype(jnp.float32)
            acc_i = i
        else:
            keep = (acc_v < v) | ((acc_v == v) & (acc_i < i))
            acc_i = jnp.where(keep, acc_i, i)
            acc_v = jnp.where(keep, acc_v, v).astype(jnp.bfloat16).astype(jnp.float32)
        base += real_sz
    idx_ref[...] = acc_i.reshape(1, 1, BN)


def _quantize_tc(flat, cb_bf, xsq, csq, cols_row):
    n_tiles = N_ROWS // BN
    idx3 = pl.pallas_call(
        _argmin_body,
        grid=(n_tiles,),
        in_specs=[
            pl.BlockSpec((BN, CODE_DIM), lambda n: (n, 0)),
            pl.BlockSpec((K_PAD_TOTAL, CODE_DIM), lambda n: (0, 0)),
            pl.BlockSpec((BN, 1), lambda n: (n, 0)),
            pl.BlockSpec((1, K_PAD_TOTAL), lambda n: (0, 0)),
            pl.BlockSpec((1, K_CHUNK_PAD), lambda n: (0, 0)),
        ],
        out_specs=pl.BlockSpec((1, 1, BN), lambda n: (n, 0, 0)),
        out_shape=jax.ShapeDtypeStruct((n_tiles, 1, BN), jnp.int32),
        compiler_params=pltpu.CompilerParams(
            dimension_semantics=("parallel",),
        ),
    )(flat, cb_bf, xsq, csq, cols_row)
    return idx3.reshape(N_ROWS)


_SC_NUM_CORES = 2
_SC_NUM_SUBCORES = 16
_NW = _SC_NUM_CORES * _SC_NUM_SUBCORES
_ROWS_PER_W = N_ROWS // _NW
_CHUNK = 128
_N_CHUNKS = _ROWS_PER_W // _CHUNK


def _gather_body(cb_hbm, idx_hbm, out_hbm, idx_v, rows_v, sem):
    wid = lax.axis_index("s") * _SC_NUM_CORES + lax.axis_index("c")
    base = wid * _ROWS_PER_W
    for c in range(_N_CHUNKS):
        off = base + c * _CHUNK
        pltpu.sync_copy(idx_hbm.at[pl.ds(off, _CHUNK)], idx_v)
        pltpu.async_copy(cb_hbm.at[idx_v], rows_v, sem).wait()
        pltpu.sync_copy(rows_v, out_hbm.at[pl.ds(off, _CHUNK)])


@functools.cache
def _gather_sc():
    return pl.kernel(
        _gather_body,
        out_type=jax.ShapeDtypeStruct((N_ROWS, CODE_DIM), jnp.float32),
        mesh=plsc.VectorSubcoreMesh(
            core_axis_name="c",
            subcore_axis_name="s",
            num_cores=_SC_NUM_CORES,
            num_subcores=_SC_NUM_SUBCORES,
        ),
        scratch_types=[
            pltpu.VMEM((_CHUNK,), jnp.int32),
            pltpu.VMEM((_CHUNK, CODE_DIM), jnp.float32),
            pltpu.SemaphoreType.DMA,
        ],
    )


def kernel(x, codebook):
    flat = x.reshape(-1, CODE_DIM)
    xsq = jnp.sum(flat * flat, axis=1, keepdims=True)
    csq = jnp.sum(codebook * codebook, axis=1)
    cb_bf = codebook.astype(jnp.bfloat16)
    cb_parts = []
    csq_parts = []
    base = 0
    for c, real_sz in enumerate(K_CHUNK_REAL):
        pad = K_CHUNK_PAD - real_sz
        cb_parts += [cb_bf[base:base + real_sz],
                     jnp.zeros((pad, CODE_DIM), jnp.bfloat16)]
        csq_parts += [csq[base:base + real_sz],
                      jnp.full((pad,), jnp.inf, jnp.float32)]
        base += real_sz
    cb_pad = jnp.concatenate(cb_parts, axis=0)
    csq_pad = jnp.concatenate(csq_parts, axis=0)
    cols_row = lax.iota(jnp.float32, K_CHUNK_PAD)[None, :]
    idx = _quantize_tc(flat, cb_pad, xsq, csq_pad[None, :], cols_row)
    codes = _gather_sc()(codebook, idx)
    return (x, codes.reshape(x.shape))


# speedup vs baseline: 1.5600x; 1.0156x over previous
import functools

import jax
import jax.numpy as jnp
from jax import lax
from jax.experimental import pallas as pl
from jax.experimental.pallas import tpu as pltpu
from jax.experimental.pallas import tpu_sc as plsc

N_CODES = 8192
CODE_DIM = 256
N_ROWS = 16384

BN = 2048
K_CHUNK_REAL = (2736, 2736, 2720)
K_CHUNK_PAD = 2816
K_PAD_TOTAL = K_CHUNK_PAD * 3


LANES = 128
ROWB = 128


def _argmin_body(x_ref, cb_ref, xsq_ref, csq_ref, cols_ref, idx_ref):
    xm2 = (-2.0 * x_ref[...]).astype(jnp.bfloat16)
    xsq = xsq_ref[...]
    acc_v = None
    acc_i = None
    base = 0
    for c, real_sz in enumerate(K_CHUNK_REAL):
        lo = c * K_CHUNK_PAD
        dots = lax.dot_general(
            xm2, cb_ref[pl.ds(lo, K_CHUNK_PAD), :],
            (((1,), (1,)), ((), ())),
            preferred_element_type=jnp.float32,
        )
        v_blocks = []
        i_blocks = []
        for r0 in range(0, BN, ROWB):
            run_v = jnp.full((ROWB, LANES), jnp.inf, jnp.float32)
            run_c = jnp.zeros((ROWB, LANES), jnp.float32)
            xsq_b = xsq[r0:r0 + ROWB, :]
            for k in range(K_CHUNK_PAD // LANES):
                s0, s1 = k * LANES, (k + 1) * LANES
                d = (xsq_b + dots[r0:r0 + ROWB, s0:s1]) \
                    + csq_ref[:, pl.ds(lo + s0, LANES)]
                m = d < run_v
                run_v = jnp.where(m, d, run_v)
                run_c = jnp.where(m, cols_ref[:, pl.ds(s0, LANES)], run_c)
            vb = jnp.min(run_v, axis=1, keepdims=True)
            ib = jnp.min(
                jnp.where(run_v == vb, run_c, jnp.float32(K_PAD_TOTAL)),
                axis=1, keepdims=True,
            )
            v_blocks.append(vb)
            i_blocks.append(ib)
        v = jnp.concatenate(v_blocks, axis=0)
        i_f = jnp.concatenate(i_blocks, axis=0)
        i = i_f.astype(jnp.int32) + base
        if acc_v is None:
            acc_v = v.astype(jnp.bfloat16).astype(jnp.float32)
            acc_i = i
        else:
            keep = (acc_v < v) | ((acc_v == v) & (acc_i < i))
            acc_i = jnp.where(keep, acc_i, i)
            acc_v = jnp.where(keep, acc_v, v).astype(jnp.bfloat16).astype(jnp.float32)
        base += real_sz
    idx_ref[...] = acc_i.reshape(1, 1, BN)


def _quantize_tc(flat, cb_bf, xsq, csq, cols_row):
    n_tiles = N_ROWS // BN
    idx3 = pl.pallas_call(
        _argmin_body,
        grid=(n_tiles,),
        in_specs=[
            pl.BlockSpec((BN, CODE_DIM), lambda n: (n, 0)),
            pl.BlockSpec((K_PAD_TOTAL, CODE_DIM), lambda n: (0, 0)),
            pl.BlockSpec((BN, 1), lambda n: (n, 0)),
            pl.BlockSpec((1, K_PAD_TOTAL), lambda n: (0, 0)),
            pl.BlockSpec((1, K_CHUNK_PAD), lambda n: (0, 0)),
        ],
        out_specs=pl.BlockSpec((1, 1, BN), lambda n: (n, 0, 0)),
        out_shape=jax.ShapeDtypeStruct((n_tiles, 1, BN), jnp.int32),
        compiler_params=pltpu.CompilerParams(
            dimension_semantics=("parallel",),
        ),
    )(flat, cb_bf, xsq, csq, cols_row)
    return idx3.reshape(N_ROWS)


_SC_NUM_CORES = 2
_SC_NUM_SUBCORES = 16
_NW = _SC_NUM_CORES * _SC_NUM_SUBCORES
_ROWS_PER_W = N_ROWS // _NW
_CHUNK = 128
_N_CHUNKS = _ROWS_PER_W // _CHUNK


def _gather_body(cb_hbm, idx_hbm, out_hbm, idx_v, rows_v, sem):
    wid = lax.axis_index("s") * _SC_NUM_CORES + lax.axis_index("c")
    base = wid * _ROWS_PER_W
    for c in range(_N_CHUNKS):
        off = base + c * _CHUNK
        pltpu.sync_copy(idx_hbm.at[pl.ds(off, _CHUNK)], idx_v)
        pltpu.async_copy(cb_hbm.at[idx_v], rows_v, sem).wait()
        pltpu.sync_copy(rows_v, out_hbm.at[pl.ds(off, _CHUNK)])


@functools.cache
def _gather_sc():
    return pl.kernel(
        _gather_body,
        out_type=jax.ShapeDtypeStruct((N_ROWS, CODE_DIM), jnp.float32),
        mesh=plsc.VectorSubcoreMesh(
            core_axis_name="c",
            subcore_axis_name="s",
            num_cores=_SC_NUM_CORES,
            num_subcores=_SC_NUM_SUBCORES,
        ),
        scratch_types=[
            pltpu.VMEM((_CHUNK,), jnp.int32),
            pltpu.VMEM((_CHUNK, CODE_DIM), jnp.float32),
            pltpu.SemaphoreType.DMA,
        ],
    )


def kernel(x, codebook):
    flat = x.reshape(-1, CODE_DIM)
    xsq = jnp.sum(flat * flat, axis=1, keepdims=True)
    csq = jnp.sum(codebook * codebook, axis=1)
    cb_bf = codebook.astype(jnp.bfloat16)
    cb_parts = []
    csq_parts = []
    base = 0
    for c, real_sz in enumerate(K_CHUNK_REAL):
        pad = K_CHUNK_PAD - real_sz
        cb_parts += [cb_bf[base:base + real_sz],
                     jnp.zeros((pad, CODE_DIM), jnp.bfloat16)]
        csq_parts += [csq[base:base + real_sz],
                      jnp.full((pad,), jnp.inf, jnp.float32)]
        base += real_sz
    cb_pad = jnp.concatenate(cb_parts, axis=0)
    csq_pad = jnp.concatenate(csq_parts, axis=0)
    cols_row = lax.iota(jnp.float32, K_CHUNK_PAD)[None, :]
    idx = _quantize_tc(flat, cb_pad, xsq, csq_pad[None, :], cols_row)
    codes = _gather_sc()(codebook, idx)
    return (x, codes.reshape(x.shape))
